# baseline (speedup 1.0000x reference)
import functools

import jax
import jax.numpy as jnp
from jax import lax
from jax.experimental import pallas as pl
from jax.experimental.pallas import tpu as pltpu
from jax.experimental.pallas import tpu_sc as plsc

B = 4096
S = 200
D = 64
NUM_CLASSES = 1000
L = 16
NC = 2
NS = 16
NW = NC * NS
NB = B // NW
CHUNK = 2
NCHUNK = NB // CHUNK
IDXW = 100
IDXROWS = CHUNK * S // IDXW
TOK = CHUNK * S
IDX_ROWS_PER_W = NB * S // IDXW


def _sc_pool(table, idx2d, cidx0, cidx1, cidx2, cat0, cat1, cat2):
    mesh = plsc.VectorSubcoreMesh(
        core_axis_name="c", subcore_axis_name="s",
        num_cores=NC, num_subcores=NS)

    @functools.partial(
        pl.kernel,
        out_type=jax.ShapeDtypeStruct((B, D), jnp.float32),
        mesh=mesh,
        compiler_params=pltpu.CompilerParams(
            needs_layout_passes=False, use_tc_tiling_on_sc=False),
        scratch_types=[
            pltpu.VMEM((IDXROWS, IDXW), jnp.int32),
            pltpu.VMEM((TOK, D), jnp.float32),
            pltpu.VMEM((NB,), jnp.int32),
            pltpu.VMEM((NB, D), jnp.float32),
            pltpu.VMEM((NB, D), jnp.float32),
            pltpu.VMEM((NB, D), jnp.float32),
            pltpu.VMEM((NB, D), jnp.float32),
            pltpu.SemaphoreType.DMA,
        ],
    )
    def k(table_h, idx_h, c0i_h, c1i_h, c2i_h, cat0_h, cat1_h, cat2_h,
          out_h, idxv, rows, cidxv, cr0, cr1, cr2, pooled, sem):
        wid = lax.axis_index("s") * NC + lax.axis_index("c")
        lane = lax.iota(jnp.int32, L)

        def seg_count(rr):
            cv = jnp.zeros((L,), jnp.int32)
            for t in range(IDXW // L):
                v = idxv[rr, pl.ds(L * t, L)]
                cv = cv + plsc.all_reduce_population_count(v != 0)
            v = idxv[rr, pl.ds(IDXW - L, L)]
            tail = IDXW % L
            cv = cv + plsc.all_reduce_population_count((lane >= L - tail) & (v != 0))
            return cv

        def row_accum(r):
            def tbody(t, a):
                a0, a1, a2, a3 = a
                base = r * S + t * 8
                for u in range(8):
                    tt = base + u
                    a0 = a0 + rows[tt, pl.ds(0, L)]
                    a1 = a1 + rows[tt, pl.ds(L, L)]
                    a2 = a2 + rows[tt, pl.ds(2 * L, L)]
                    a3 = a3 + rows[tt, pl.ds(3 * L, L)]
                return (a0, a1, a2, a3)
            z = jnp.zeros((L,), jnp.float32)
            return lax.fori_loop(0, S // 8, tbody, (z, z, z, z))

        def chunk(i, carry):
            ib = wid * IDX_ROWS_PER_W + i * IDXROWS
            pltpu.sync_copy(idx_h.at[pl.ds(ib, IDXROWS)], idxv)
            cps = [
                pltpu.async_copy(
                    table_h.at[idxv.at[j]],
                    rows.at[pl.ds(j * IDXW, IDXW)], sem)
                for j in range(IDXROWS)
            ]
            for cp in cps:
                cp.wait()
            for r in range(CHUNK):
                a0, a1, a2, a3 = row_accum(r)
                cv = seg_count(2 * r) + seg_count(2 * r + 1)
                inv = jnp.where(cv > 0, 1.0 / cv.astype(jnp.float32), 0.0)
                row = i * CHUNK + r
                pooled[row, pl.ds(0, L)] = a0 * inv
                pooled[row, pl.ds(L, L)] = a1 * inv
                pooled[row, pl.ds(2 * L, L)] = a2 * inv
                pooled[row, pl.ds(3 * L, L)] = a3 * inv
            return carry

        lax.fori_loop(0, NCHUNK, chunk, 0)

        base = wid * NB
        pltpu.sync_copy(c0i_h.at[pl.ds(base, NB)], cidxv)
        pltpu.async_copy(cat0_h.at[cidxv], cr0, sem).wait()
        pltpu.sync_copy(c1i_h.at[pl.ds(base, NB)], cidxv)
        pltpu.async_copy(cat1_h.at[cidxv], cr1, sem).wait()
        pltpu.sync_copy(c2i_h.at[pl.ds(base, NB)], cidxv)
        pltpu.async_copy(cat2_h.at[cidxv], cr2, sem).wait()

        def cbody(r, carry):
            for j in range(D // L):
                sl = pl.ds(L * j, L)
                pooled[r, sl] = pooled[r, sl] + cr0[r, sl] + cr1[r, sl] + cr2[r, sl]
            return carry

        lax.fori_loop(0, NB, cbody, 0)
        pltpu.sync_copy(pooled, out_h.at[pl.ds(base, NB)])

    return k(table, idx2d, cidx0, cidx1, cidx2, cat0, cat1, cat2)


def _linear(x, W, b):
    BM = 512

    def mm(x_ref, w_ref, b_ref, o_ref):
        o_ref[...] = lax.dot_general(
            x_ref[...], w_ref[...], (((1,), (1,)), ((), ())),
            preferred_element_type=jnp.float32) + b_ref[...]

    return pl.pallas_call(
        mm,
        grid=(B // BM,),
        in_specs=[
            pl.BlockSpec((BM, D), lambda i: (i, 0)),
            pl.BlockSpec((NUM_CLASSES, D), lambda i: (0, 0)),
            pl.BlockSpec((1, NUM_CLASSES), lambda i: (0, 0)),
        ],
        out_specs=pl.BlockSpec((BM, NUM_CLASSES), lambda i: (i, 0)),
        out_shape=jax.ShapeDtypeStruct((B, NUM_CLASSES), jnp.float32),
    )(x, W, b.reshape(1, NUM_CLASSES))


def kernel(encoded_text, additional_inputs, emb_table, cat0, cat1, cat2, W, b):
    idx2d = encoded_text.reshape(B * S // IDXW, IDXW)
    cidx0 = additional_inputs[:, 0]
    cidx1 = additional_inputs[:, 1]
    cidx2 = additional_inputs[:, 2]
    pooled = _sc_pool(emb_table, idx2d, cidx0, cidx1, cidx2, cat0, cat1, cat2)
    return _linear(pooled, W, b)


# speedup vs baseline: 1.3982x; 1.3982x over previous
import functools

import jax
import jax.numpy as jnp
from jax import lax
from jax.experimental import pallas as pl
from jax.experimental.pallas import tpu as pltpu
from jax.experimental.pallas import tpu_sc as plsc

VOCAB = 1000000
HALF = 1 << 19
B = 4096
S = 200
D = 64
NUM_CLASSES = 1000
L = 16
NC = 2
NS = 16
NW = NC * NS
NB = B // NW
CHUNK = 2
NCHUNK = NB // CHUNK
IDXW = 80
NSTREAM = CHUNK * S // IDXW
TOK = CHUNK * S


def _format_table(tableT):
    BN = 2048
    ngrid = HALF // BN

    def tr(a_ref, b_ref, o_ref):
        o_ref[:, 0:D] = a_ref[...].T
        o_ref[:, D:2 * D] = b_ref[...].T

    last = pl.cdiv(VOCAB, BN) - 1

    return pl.pallas_call(
        tr,
        grid=(ngrid,),
        in_specs=[
            pl.BlockSpec((D, BN), lambda i: (0, i)),
            pl.BlockSpec((D, BN), lambda i: (0, jnp.minimum(i + ngrid, last))),
        ],
        out_specs=pl.BlockSpec((BN, 2 * D), lambda i: (i, 0)),
        out_shape=jax.ShapeDtypeStruct((HALF, 2 * D), jnp.float32),
    )(tableT, tableT)


def _sc_pool(table2, idx1d, cidx0, cidx1, cidx2, cat0, cat1, cat2):
    mesh = plsc.VectorSubcoreMesh(
        core_axis_name="c", subcore_axis_name="s",
        num_cores=NC, num_subcores=NS)

    @functools.partial(
        pl.kernel,
        out_type=jax.ShapeDtypeStruct((B, D), jnp.float32),
        mesh=mesh,
        compiler_params=pltpu.CompilerParams(
            needs_layout_passes=False, use_tc_tiling_on_sc=False),
        scratch_types=[
            pltpu.VMEM((TOK,), jnp.int32),
            pltpu.VMEM((TOK, D), jnp.float32),
            pltpu.VMEM((NB,), jnp.int32),
            pltpu.VMEM((NB, D), jnp.float32),
            pltpu.VMEM((NB, D), jnp.float32),
            pltpu.VMEM((NB, D), jnp.float32),
            pltpu.VMEM((NB, D), jnp.float32),
            pltpu.SemaphoreType.DMA,
        ],
    )
    def k(table_h, idx_h, c0i_h, c1i_h, c2i_h, cat0_h, cat1_h, cat2_h,
          out_h, idxv, rows, cidxv, cr0, cr1, cr2, pooled, sem):
        wid = lax.axis_index("s") * NC + lax.axis_index("c")
        lane = lax.iota(jnp.int32, L)

        def seg_count(r):
            cv = jnp.zeros((L,), jnp.int32)
            for t in range(S // L):
                v = idxv[pl.ds(r * S + L * t, L)]
                cv = cv + plsc.all_reduce_population_count(v != 0)
            tail = S % L
            v = idxv[pl.ds(r * S + S - L, L)]
            cv = cv + plsc.all_reduce_population_count((lane >= L - tail) & (v != 0))
            return cv

        def row_accum(r):
            def tbody(t, a):
                a0, a1, a2, a3 = a
                base = r * S + t * 8
                for u in range(8):
                    tt = base + u
                    a0 = a0 + rows[tt, pl.ds(0, L)]
                    a1 = a1 + rows[tt, pl.ds(L, L)]
                    a2 = a2 + rows[tt, pl.ds(2 * L, L)]
                    a3 = a3 + rows[tt, pl.ds(3 * L, L)]
                return (a0, a1, a2, a3)
            z = jnp.zeros((L,), jnp.float32)
            return lax.fori_loop(0, S // 8, tbody, (z, z, z, z))

        def chunk(i, carry):
            ib = wid * (NB * S) + i * TOK
            pltpu.sync_copy(idx_h.at[pl.ds(ib, TOK)], idxv)
            for t in range(TOK // L):
                v = idxv[pl.ds(L * t, L)]
                idxv[pl.ds(L * t, L)] = ((v << 1) | (v >> 19)) & (2 * HALF - 1)
            cps = [
                pltpu.async_copy(
                    table_h.at[idxv.at[pl.ds(j * IDXW, IDXW)]],
                    rows.at[pl.ds(j * IDXW, IDXW)], sem)
                for j in range(NSTREAM)
            ]
            for cp in cps:
                cp.wait()
            for r in range(CHUNK):
                a0, a1, a2, a3 = row_accum(r)
                cv = seg_count(r)
                inv = jnp.where(cv > 0, 1.0 / cv.astype(jnp.float32), 0.0)
                row = i * CHUNK + r
                pooled[row, pl.ds(0, L)] = a0 * inv
                pooled[row, pl.ds(L, L)] = a1 * inv
                pooled[row, pl.ds(2 * L, L)] = a2 * inv
                pooled[row, pl.ds(3 * L, L)] = a3 * inv
            return carry

        lax.fori_loop(0, NCHUNK, chunk, 0)

        base = wid * NB
        pltpu.sync_copy(c0i_h.at[pl.ds(base, NB)], cidxv)
        pltpu.async_copy(cat0_h.at[cidxv], cr0, sem).wait()
        pltpu.sync_copy(c1i_h.at[pl.ds(base, NB)], cidxv)
        pltpu.async_copy(cat1_h.at[cidxv], cr1, sem).wait()
        pltpu.sync_copy(c2i_h.at[pl.ds(base, NB)], cidxv)
        pltpu.async_copy(cat2_h.at[cidxv], cr2, sem).wait()

        def cbody(r, carry):
            for j in range(D // L):
                sl = pl.ds(L * j, L)
                pooled[r, sl] = pooled[r, sl] + cr0[r, sl] + cr1[r, sl] + cr2[r, sl]
            return carry

        lax.fori_loop(0, NB, cbody, 0)
        pltpu.sync_copy(pooled, out_h.at[pl.ds(base, NB)])

    return k(table2, idx1d, cidx0, cidx1, cidx2, cat0, cat1, cat2)


def _linear(x, W, b):
    BM = 512

    def mm(x_ref, w_ref, b_ref, o_ref):
        o_ref[...] = lax.dot_general(
            x_ref[...], w_ref[...], (((1,), (1,)), ((), ())),
            preferred_element_type=jnp.float32) + b_ref[...]

    return pl.pallas_call(
        mm,
        grid=(B // BM,),
        in_specs=[
            pl.BlockSpec((BM, D), lambda i: (i, 0)),
            pl.BlockSpec((NUM_CLASSES, D), lambda i: (0, 0)),
            pl.BlockSpec((1, NUM_CLASSES), lambda i: (0, 0)),
        ],
        out_specs=pl.BlockSpec((BM, NUM_CLASSES), lambda i: (i, 0)),
        out_shape=jax.ShapeDtypeStruct((B, NUM_CLASSES), jnp.float32),
    )(x, W, b.reshape(1, NUM_CLASSES))


def kernel(encoded_text, additional_inputs, emb_table, cat0, cat1, cat2, W, b):
    t128 = _format_table(emb_table.T)
    table2 = t128.reshape(2 * HALF, D)
    idx1d = encoded_text.reshape(B * S)
    cidx0 = additional_inputs[:, 0]
    cidx1 = additional_inputs[:, 1]
    cidx2 = additional_inputs[:, 2]
    pooled = _sc_pool(table2, idx1d, cidx0, cidx1, cidx2, cat0, cat1, cat2)
    return _linear(pooled, W, b)
